# baseline (speedup 1.0000x reference)
import functools

import jax
import jax.numpy as jnp
from jax import lax
from jax.experimental import pallas as pl
from jax.experimental.pallas import tpu as pltpu
from jax.experimental.pallas import tpu_sc as plsc

EMBED_DIM = 64
BATCH = 16384
CHUNK = 32

_info = plsc.get_sparse_core_info()
_NC, _NS = _info.num_cores, _info.num_subcores
_NW = _NC * _NS


def _make_gather(dim, batch, num_pairs):
    b_per_w = batch // _NW
    n_chunks = b_per_w // CHUNK
    pair_w = 2 * dim
    mesh = plsc.VectorSubcoreMesh(core_axis_name="c", subcore_axis_name="s")

    @functools.partial(
        pl.kernel,
        mesh=mesh,
        out_type=jax.ShapeDtypeStruct((batch, dim), jnp.float32),
        scratch_types=[
            pltpu.VMEM((b_per_w,), jnp.int32),
            pltpu.VMEM((b_per_w,), jnp.int32),
            pltpu.VMEM((b_per_w,), jnp.int32),
            pltpu.VMEM((CHUNK, pair_w), jnp.float32),
            pltpu.VMEM((CHUNK, pair_w), jnp.float32),
            pltpu.VMEM((CHUNK, dim), jnp.float32),
            pltpu.VMEM((CHUNK, dim), jnp.float32),
            pltpu.SemaphoreType.DMA,
            pltpu.SemaphoreType.DMA,
        ],
    )
    def gather_kernel(idx_hbm, t128_hbm, out_hbm, idx_v, p_v, h_v,
                      tiles0, tiles1, rows0, rows1, sem0, sem1):
        wid = lax.axis_index("s") * _NC + lax.axis_index("c")
        base = wid * b_per_w
        pltpu.sync_copy(idx_hbm.at[pl.ds(base, b_per_w)], idx_v)

        def index_math(g, carry):
            v = idx_v[pl.ds(g * 16, 16)]
            p_v[pl.ds(g * 16, 16)] = lax.shift_right_logical(v, 1)
            h_v[pl.ds(g * 16, 16)] = lax.bitwise_and(v, 1)
            return carry

        lax.fori_loop(0, b_per_w // 16, index_math, 0)

        def fire(c, tiles, sem):
            return pltpu.async_copy(
                t128_hbm.at[p_v.at[pl.ds(c * CHUNK, CHUNK)]], tiles, sem
            )

        def select(c, tiles, rows):
            for g2 in range(CHUNK // 16):
                hvec = h_v[pl.ds(c * CHUNK + g2 * 16, 16)]
                for l in range(16):
                    j = g2 * 16 + l
                    off = hvec[l] * dim
                    for kk in range(dim // 16):
                        rows[j, pl.ds(kk * 16, 16)] = (
                            tiles[j, pl.ds(off + kk * 16, 16)]
                        )
            pltpu.sync_copy(
                rows, out_hbm.at[pl.ds(base + c * CHUNK, CHUNK)]
            )

        def pair_loop(p, carry):
            c0 = p * 2
            c1 = c0 + 1
            d0 = fire(c0, tiles0, sem0)
            d1 = fire(c1, tiles1, sem1)
            d0.wait()
            select(c0, tiles0, rows0)
            d1.wait()
            select(c1, tiles1, rows1)
            return carry

        lax.fori_loop(0, n_chunks // 2, pair_loop, 0)

    return gather_kernel


@jax.jit
def kernel(user_inputs, table):
    num_rows, dim = table.shape
    t128 = table.reshape(num_rows // 2, 2 * dim)
    gather = _make_gather(dim, user_inputs.shape[0], num_rows // 2)
    return gather(user_inputs.astype(jnp.int32), t128)
